# baseline (speedup 1.0000x reference)
import functools

import jax
import jax.numpy as jnp
from jax import lax
from jax.experimental import pallas as pl
from jax.experimental.pallas import tpu as pltpu
from jax.experimental.pallas import tpu_sc as plsc


def _sc_gather_rows(table, ids):
    info = plsc.get_sparse_core_info()
    nc, ns = info.num_cores, info.num_subcores
    nw = nc * ns
    b = ids.shape[0]
    d = table.shape[1]
    b_per_w = b // nw
    mesh = plsc.VectorSubcoreMesh(core_axis_name="c", subcore_axis_name="s")

    @functools.partial(
        pl.kernel,
        mesh=mesh,
        out_type=jax.ShapeDtypeStruct((b, d), table.dtype),
        scratch_types=[
            pltpu.VMEM((b_per_w,), jnp.int32),
            pltpu.VMEM((b_per_w, d), table.dtype),
            pltpu.SemaphoreType.DMA,
        ],
    )
    def gather(table_hbm, idx_hbm, out_hbm, idx_v, rows_v, sem):
        wid = lax.axis_index("s") * nc + lax.axis_index("c")
        base = wid * b_per_w
        pltpu.sync_copy(idx_hbm.at[pl.ds(base, b_per_w)], idx_v)
        pltpu.async_copy(table_hbm.at[idx_v], rows_v, sem).wait()
        pltpu.sync_copy(rows_v, out_hbm.at[pl.ds(base, b_per_w)])

    return gather(table, ids)


def _add_body(x_ref, bias_ref, o_ref):
    o_ref[...] = x_ref[...] + bias_ref[...][:, None, :]


def _tc_add_bias_t(xt, bias, bb=128):
    b, t, c = xt.shape
    return pl.pallas_call(
        _add_body,
        grid=(b // bb,),
        in_specs=[
            pl.BlockSpec((bb, t, c), lambda i: (i, 0, 0)),
            pl.BlockSpec((bb, c), lambda i: (i, 0)),
        ],
        out_specs=pl.BlockSpec((bb, t, c), lambda i: (i, 0, 0)),
        out_shape=jax.ShapeDtypeStruct((b, t, c), xt.dtype),
    )(xt, bias)


def kernel(x, subject_ids, table):
    ids = subject_ids.astype(jnp.int32)
    bias = _sc_gather_rows(table, ids)
    xt = jnp.transpose(x, (0, 2, 1))
    out_t = _tc_add_bias_t(xt, bias)
    return jnp.transpose(out_t, (0, 2, 1))


# speedup vs baseline: 1.0051x; 1.0051x over previous
import functools

import jax
import jax.numpy as jnp
from jax import lax
from jax.experimental import pallas as pl
from jax.experimental.pallas import tpu as pltpu
from jax.experimental.pallas import tpu_sc as plsc


def _sc_gather_rows(table, ids):
    info = plsc.get_sparse_core_info()
    nc, ns = info.num_cores, info.num_subcores
    nw = nc * ns
    b = ids.shape[0]
    d = table.shape[1]
    b_per_w = b // nw
    mesh = plsc.VectorSubcoreMesh(core_axis_name="c", subcore_axis_name="s", num_cores=1)

    @functools.partial(
        pl.kernel,
        mesh=mesh,
        out_type=jax.ShapeDtypeStruct((b, d), table.dtype),
        scratch_types=[
            pltpu.VMEM((b_per_w,), jnp.int32),
            pltpu.VMEM((b_per_w, d), table.dtype),
            pltpu.SemaphoreType.DMA,
        ],
    )
    def gather(table_hbm, idx_hbm, out_hbm, idx_v, rows_v, sem):
        wid = lax.axis_index("s") * nc + lax.axis_index("c")
        base = wid * b_per_w
        pltpu.sync_copy(idx_hbm.at[pl.ds(base, b_per_w)], idx_v)
        pltpu.async_copy(table_hbm.at[idx_v], rows_v, sem).wait()
        pltpu.sync_copy(rows_v, out_hbm.at[pl.ds(base, b_per_w)])

    return gather(table, ids)


def _add_body(x_ref, bias_ref, o_ref):
    o_ref[...] = x_ref[...] + bias_ref[...][:, None, :]


def _tc_add_bias_t(xt, bias, bb=128):
    b, t, c = xt.shape
    return pl.pallas_call(
        _add_body,
        grid=(b // bb,),
        in_specs=[
            pl.BlockSpec((bb, t, c), lambda i: (i, 0, 0)),
            pl.BlockSpec((bb, c), lambda i: (i, 0)),
        ],
        out_specs=pl.BlockSpec((bb, t, c), lambda i: (i, 0, 0)),
        out_shape=jax.ShapeDtypeStruct((b, t, c), xt.dtype),
    )(xt, bias)


def kernel(x, subject_ids, table):
    ids = subject_ids.astype(jnp.int32)
    bias = _sc_gather_rows(table, ids)
    xt = jnp.transpose(x, (0, 2, 1))
    out_t = _tc_add_bias_t(xt, bias)
    return jnp.transpose(out_t, (0, 2, 1))
